# baseline (speedup 1.0000x reference)
import jax
import jax.numpy as jnp
from jax import lax
from jax.experimental import pallas as pl
from jax.experimental.pallas import tpu as pltpu
from jax.experimental.pallas import tpu_sc as plsc

APP_DIM = 32
B = 4096
L_INSTALL = 200
L_ECC = 50
NC = 2
NS = 16
NW = NC * NS
BPW = B // NW
NACC_I = 4
NACC_E = 5


def _body(inst_t_hbm, ecc_t_hbm, table_hbm, out_hbm,
          slab_i, slab_e, acc_i, acc_e, out_v, *sems):
    sems_i = sems[:NACC_I]
    sems_e = sems[NACC_I:]
    wid = lax.axis_index("s") * NC + lax.axis_index("c")
    base = wid * BPW

    pltpu.sync_copy(inst_t_hbm.at[:, pl.ds(base, BPW)], slab_i)
    pltpu.sync_copy(ecc_t_hbm.at[:, pl.ds(base, BPW)], slab_e)

    for k in range(NACC_I):
        pltpu.async_copy(table_hbm.at[slab_i.at[k]], acc_i.at[k], sems_i[k])

    def inst_group(g, carry):
        for k in range(NACC_I):
            j = g * NACC_I + k
            pltpu.make_async_copy(table_hbm.at[slab_i.at[j]], acc_i.at[k],
                                  sems_i[k]).wait()
            pltpu.async_copy(table_hbm.at[slab_i.at[j]], acc_i.at[k],
                             sems_i[k], add=True)
        return carry

    lax.fori_loop(1, L_INSTALL // NACC_I, inst_group, 0)
    for k in range(NACC_I):
        pltpu.make_async_copy(table_hbm.at[slab_i.at[k]], acc_i.at[k],
                              sems_i[k]).wait()

    for k in range(NACC_E):
        pltpu.async_copy(table_hbm.at[slab_e.at[k]], acc_e.at[k], sems_e[k])

    def ecc_group(g, carry):
        for k in range(NACC_E):
            j = g * NACC_E + k
            pltpu.make_async_copy(table_hbm.at[slab_e.at[j]], acc_e.at[k],
                                  sems_e[k]).wait()
            pltpu.async_copy(table_hbm.at[slab_e.at[j]], acc_e.at[k],
                             sems_e[k], add=True)
        return carry

    lax.fori_loop(1, L_ECC // NACC_E, ecc_group, 0)

    def comb_i(i, carry):
        for c in range(2):
            s = (acc_i[0, i, pl.ds(16 * c, 16)] + acc_i[1, i, pl.ds(16 * c, 16)]
                 + acc_i[2, i, pl.ds(16 * c, 16)] + acc_i[3, i, pl.ds(16 * c, 16)])
            out_v[i, pl.ds(16 * c, 16)] = s * (1.0 / L_INSTALL)
        return carry

    lax.fori_loop(0, BPW, comb_i, 0)

    for k in range(NACC_E):
        pltpu.make_async_copy(table_hbm.at[slab_e.at[k]], acc_e.at[k],
                              sems_e[k]).wait()

    def comb_e(i, carry):
        for c in range(2):
            s = (acc_e[0, i, pl.ds(16 * c, 16)] + acc_e[1, i, pl.ds(16 * c, 16)]
                 + acc_e[2, i, pl.ds(16 * c, 16)] + acc_e[3, i, pl.ds(16 * c, 16)]
                 + acc_e[4, i, pl.ds(16 * c, 16)])
            out_v[i, pl.ds(32 + 16 * c, 16)] = s * (1.0 / L_ECC)
        return carry

    lax.fori_loop(0, BPW, comb_e, 0)
    pltpu.sync_copy(out_v, out_hbm.at[pl.ds(base, BPW)])


@jax.jit
def kernel(install, install_ecc, app_table):
    inst_t = install.astype(jnp.int32).T
    ecc_t = install_ecc.astype(jnp.int32).T
    mesh = plsc.VectorSubcoreMesh(core_axis_name="c", subcore_axis_name="s")
    run = pl.kernel(
        _body,
        mesh=mesh,
        out_type=jax.ShapeDtypeStruct((B, 2 * APP_DIM), jnp.float32),
        scratch_types=[
            pltpu.VMEM((L_INSTALL, BPW), jnp.int32),
            pltpu.VMEM((L_ECC, BPW), jnp.int32),
            pltpu.VMEM((NACC_I, BPW, APP_DIM), jnp.float32),
            pltpu.VMEM((NACC_E, BPW, APP_DIM), jnp.float32),
            pltpu.VMEM((BPW, 2 * APP_DIM), jnp.float32),
        ] + [pltpu.SemaphoreType.DMA] * (NACC_I + NACC_E),
        compiler_params=pltpu.CompilerParams(use_tc_tiling_on_sc=False),
    )
    return run(inst_t, ecc_t, app_table)


# speedup vs baseline: 1.1841x; 1.1841x over previous
import jax
import jax.numpy as jnp
from jax import lax
from jax.experimental import pallas as pl
from jax.experimental.pallas import tpu as pltpu
from jax.experimental.pallas import tpu_sc as plsc

APP_SIZE_PAD = 1001472
B = 4096
APP_DIM = 32
L_INSTALL = 200
L_ECC = 50
NC = 2
NS = 16
NW = NC * NS
BPW = B // NW
NACC_I = 8
NACC_E = 5

PACK_W = 2048
PACK_GRID = 489


def _tc_pack_body(x_ref, o_ref):
    x = x_ref[...]
    col_groups = []
    for a in range(4):
        rows = [jnp.transpose(x[:, 128 * (4 * s + a):128 * (4 * s + a) + 128])
                for s in range(4)]
        col_groups.append(jnp.concatenate(rows, axis=0))
    o_ref[...] = jnp.concatenate(col_groups, axis=1)


def _sc_body(inst_t_hbm, ecc_t_hbm, table_hbm, out_hbm,
             slab_i, slab_e, acc_i, acc_e, out_v, *sems):
    sems_i = sems[:NACC_I]
    sems_e = sems[NACC_I:]
    wid = lax.axis_index("s") * NC + lax.axis_index("c")
    base = wid * BPW

    pltpu.sync_copy(inst_t_hbm.at[:, pl.ds(base, BPW)], slab_i)
    pltpu.sync_copy(ecc_t_hbm.at[:, pl.ds(base, BPW)], slab_e)

    for k in range(NACC_I):
        pltpu.async_copy(table_hbm.at[slab_i.at[k]], acc_i.at[k], sems_i[k])

    def inst_group(g, carry):
        for k in range(NACC_I):
            j = g * NACC_I + k
            pltpu.make_async_copy(table_hbm.at[slab_i.at[j]], acc_i.at[k],
                                  sems_i[k]).wait()
            pltpu.async_copy(table_hbm.at[slab_i.at[j]], acc_i.at[k],
                             sems_i[k], add=True)
        return carry

    lax.fori_loop(1, L_INSTALL // NACC_I, inst_group, 0)
    for k in range(NACC_I):
        pltpu.make_async_copy(table_hbm.at[slab_i.at[k]], acc_i.at[k],
                              sems_i[k]).wait()

    for k in range(NACC_E):
        pltpu.async_copy(table_hbm.at[slab_e.at[k]], acc_e.at[k], sems_e[k])

    def ecc_group(g, carry):
        for k in range(NACC_E):
            j = g * NACC_E + k
            pltpu.make_async_copy(table_hbm.at[slab_e.at[j]], acc_e.at[k],
                                  sems_e[k]).wait()
            pltpu.async_copy(table_hbm.at[slab_e.at[j]], acc_e.at[k],
                             sems_e[k], add=True)
        return carry

    lax.fori_loop(1, L_ECC // NACC_E, ecc_group, 0)

    def comb_i(i, carry):
        for c in range(2):
            s = acc_i[0, i, pl.ds(16 * c, 16)]
            for k in range(1, NACC_I):
                s = s + acc_i[k, i, pl.ds(16 * c, 16)]
            out_v[i, pl.ds(16 * c, 16)] = s * (1.0 / L_INSTALL)
        return carry

    lax.fori_loop(0, BPW, comb_i, 0)

    for k in range(NACC_E):
        pltpu.make_async_copy(table_hbm.at[slab_e.at[k]], acc_e.at[k],
                              sems_e[k]).wait()

    def comb_e(i, carry):
        for c in range(2):
            s = acc_e[0, i, pl.ds(16 * c, 16)]
            for k in range(1, NACC_E):
                s = s + acc_e[k, i, pl.ds(16 * c, 16)]
            out_v[i, pl.ds(32 + 16 * c, 16)] = s * (1.0 / L_ECC)
        return carry

    lax.fori_loop(0, BPW, comb_e, 0)
    pltpu.sync_copy(out_v, out_hbm.at[pl.ds(base, BPW)])


def _permute_rows(r):
    return (r & ~jnp.int32(511)) + ((r & 127) << 2) + ((r >> 7) & 3)


@jax.jit
def kernel(install, install_ecc, app_table):
    inst_t = _permute_rows(install.astype(jnp.int32).T)
    ecc_t = _permute_rows(install_ecc.astype(jnp.int32).T)

    packed = pl.pallas_call(
        _tc_pack_body,
        grid=(PACK_GRID,),
        in_specs=[pl.BlockSpec((APP_DIM, PACK_W), lambda j: (0, j))],
        out_specs=pl.BlockSpec((4 * PACK_W // 16, 128), lambda j: (j, 0)),
        out_shape=jax.ShapeDtypeStruct((PACK_GRID * PACK_W // 4, 128),
                                       jnp.float32),
    )(app_table.T)
    tbl = packed.reshape(APP_SIZE_PAD, APP_DIM)

    mesh = plsc.VectorSubcoreMesh(core_axis_name="c", subcore_axis_name="s")
    run = pl.kernel(
        _sc_body,
        mesh=mesh,
        out_type=jax.ShapeDtypeStruct((B, 2 * APP_DIM), jnp.float32),
        scratch_types=[
            pltpu.VMEM((L_INSTALL, BPW), jnp.int32),
            pltpu.VMEM((L_ECC, BPW), jnp.int32),
            pltpu.VMEM((NACC_I, BPW, APP_DIM), jnp.float32),
            pltpu.VMEM((NACC_E, BPW, APP_DIM), jnp.float32),
            pltpu.VMEM((BPW, 2 * APP_DIM), jnp.float32),
        ] + [pltpu.SemaphoreType.DMA] * (NACC_I + NACC_E),
        compiler_params=pltpu.CompilerParams(use_tc_tiling_on_sc=False),
    )
    return run(inst_t, ecc_t, tbl)


# speedup vs baseline: 1.4438x; 1.2194x over previous
import jax
import jax.numpy as jnp
from jax import lax
from jax.experimental import pallas as pl
from jax.experimental.pallas import tpu as pltpu
from jax.experimental.pallas import tpu_sc as plsc

APP_SIZE_PAD = 1001472
B = 4096
APP_DIM = 32
L_INSTALL = 200
L_ECC = 50
NC = 2
NS = 16
NW = NC * NS
BPW = B // NW
NACC_I = 8
NACC_E = 5

PACK_W = 2048
PACK_GRID = 489


def _tc_pack_body(x_ref, o_ref):
    x = x_ref[...]
    for s in range(PACK_W // 512):
        y = jnp.concatenate(
            [x[:, 128 * (4 * s + a):128 * (4 * s + a) + 128] for a in range(4)],
            axis=0)
        o_ref[pl.ds(128 * s, 128), :] = jnp.transpose(y)


def _sc_body(inst_t_hbm, ecc_t_hbm, table_hbm, out_hbm,
             slab_i, slab_e, acc_i, acc_e, out_v, *sems):
    sems_i = sems[:NACC_I]
    sems_e = sems[NACC_I:]
    wid = lax.axis_index("s") * NC + lax.axis_index("c")
    base = wid * BPW

    pltpu.sync_copy(inst_t_hbm.at[:, pl.ds(base, BPW)], slab_i)
    pltpu.sync_copy(ecc_t_hbm.at[:, pl.ds(base, BPW)], slab_e)

    for k in range(NACC_I):
        pltpu.async_copy(table_hbm.at[slab_i.at[k]], acc_i.at[k], sems_i[k])

    def inst_group(g, carry):
        for k in range(NACC_I):
            j = g * NACC_I + k
            pltpu.make_async_copy(table_hbm.at[slab_i.at[j]], acc_i.at[k],
                                  sems_i[k]).wait()
            pltpu.async_copy(table_hbm.at[slab_i.at[j]], acc_i.at[k],
                             sems_i[k], add=True)
        return carry

    lax.fori_loop(1, L_INSTALL // NACC_I, inst_group, 0)
    for k in range(NACC_I):
        pltpu.make_async_copy(table_hbm.at[slab_i.at[k]], acc_i.at[k],
                              sems_i[k]).wait()

    for k in range(NACC_E):
        pltpu.async_copy(table_hbm.at[slab_e.at[k]], acc_e.at[k], sems_e[k])

    def ecc_group(g, carry):
        for k in range(NACC_E):
            j = g * NACC_E + k
            pltpu.make_async_copy(table_hbm.at[slab_e.at[j]], acc_e.at[k],
                                  sems_e[k]).wait()
            pltpu.async_copy(table_hbm.at[slab_e.at[j]], acc_e.at[k],
                             sems_e[k], add=True)
        return carry

    lax.fori_loop(1, L_ECC // NACC_E, ecc_group, 0)

    def comb_i(i, carry):
        for c in range(2):
            s = acc_i[0, i, pl.ds(16 * c, 16)]
            for k in range(1, NACC_I):
                s = s + acc_i[k, i, pl.ds(16 * c, 16)]
            out_v[i, pl.ds(16 * c, 16)] = s * (1.0 / L_INSTALL)
        return carry

    lax.fori_loop(0, BPW, comb_i, 0)

    for k in range(NACC_E):
        pltpu.make_async_copy(table_hbm.at[slab_e.at[k]], acc_e.at[k],
                              sems_e[k]).wait()

    def comb_e(i, carry):
        for c in range(2):
            s = acc_e[0, i, pl.ds(16 * c, 16)]
            for k in range(1, NACC_E):
                s = s + acc_e[k, i, pl.ds(16 * c, 16)]
            out_v[i, pl.ds(32 + 16 * c, 16)] = s * (1.0 / L_ECC)
        return carry

    lax.fori_loop(0, BPW, comb_e, 0)
    pltpu.sync_copy(out_v, out_hbm.at[pl.ds(base, BPW)])


def _permute_rows(r):
    return (r & ~jnp.int32(511)) + ((r & 127) << 2) + ((r >> 7) & 3)


@jax.jit
def kernel(install, install_ecc, app_table):
    inst_t = _permute_rows(install.astype(jnp.int32).T)
    ecc_t = _permute_rows(install_ecc.astype(jnp.int32).T)

    packed = pl.pallas_call(
        _tc_pack_body,
        grid=(PACK_GRID,),
        in_specs=[pl.BlockSpec((APP_DIM, PACK_W), lambda j: (0, j))],
        out_specs=pl.BlockSpec((4 * PACK_W // 16, 128), lambda j: (j, 0)),
        out_shape=jax.ShapeDtypeStruct((PACK_GRID * PACK_W // 4, 128),
                                       jnp.float32),
    )(app_table.T)
    tbl = packed.reshape(APP_SIZE_PAD, APP_DIM)

    mesh = plsc.VectorSubcoreMesh(core_axis_name="c", subcore_axis_name="s")
    run = pl.kernel(
        _sc_body,
        mesh=mesh,
        out_type=jax.ShapeDtypeStruct((B, 2 * APP_DIM), jnp.float32),
        scratch_types=[
            pltpu.VMEM((L_INSTALL, BPW), jnp.int32),
            pltpu.VMEM((L_ECC, BPW), jnp.int32),
            pltpu.VMEM((NACC_I, BPW, APP_DIM), jnp.float32),
            pltpu.VMEM((NACC_E, BPW, APP_DIM), jnp.float32),
            pltpu.VMEM((BPW, 2 * APP_DIM), jnp.float32),
        ] + [pltpu.SemaphoreType.DMA] * (NACC_I + NACC_E),
        compiler_params=pltpu.CompilerParams(use_tc_tiling_on_sc=False),
    )
    return run(inst_t, ecc_t, tbl)


# speedup vs baseline: 2.6228x; 1.8166x over previous
import jax
import jax.numpy as jnp
from jax import lax
from jax.experimental import pallas as pl
from jax.experimental.pallas import tpu as pltpu
from jax.experimental.pallas import tpu_sc as plsc

B = 4096
APP_DIM = 32
L_INSTALL = 200
L_ECC = 50
NC = 2
NS = 16
NW = NC * NS
BPW = B // NW
NACC_I = 8
NACC_E = 5

PACK_W = 8192
PACK_GRID = 123
APP_SIZE_PAD = PACK_GRID * PACK_W


def _tc_pack_body(x_ref, o_ref):
    x = x_ref[...]
    for s in range(PACK_W // 512):
        y = jnp.concatenate(
            [x[:, 128 * (4 * s + a):128 * (4 * s + a) + 128] for a in range(4)],
            axis=0)
        o_ref[pl.ds(128 * s, 128), :] = jnp.transpose(y)


def _sc_body(inst_t_hbm, ecc_t_hbm, table_hbm, out_hbm,
             slab_i, slab_e, acc_i, acc_e, out_v, *sems):
    sems_i = sems[:NACC_I]
    sems_e = sems[NACC_I:]
    wid = lax.axis_index("s") * NC + lax.axis_index("c")
    base = wid * BPW

    pltpu.sync_copy(inst_t_hbm.at[:, pl.ds(base, BPW)], slab_i)
    pltpu.sync_copy(ecc_t_hbm.at[:, pl.ds(base, BPW)], slab_e)

    for k in range(NACC_I):
        pltpu.async_copy(table_hbm.at[slab_i.at[k]], acc_i.at[k], sems_i[k])

    def inst_group(g, carry):
        for k in range(NACC_I):
            j = g * NACC_I + k
            pltpu.make_async_copy(table_hbm.at[slab_i.at[j]], acc_i.at[k],
                                  sems_i[k]).wait()
            pltpu.async_copy(table_hbm.at[slab_i.at[j]], acc_i.at[k],
                             sems_i[k], add=True)
        return carry

    lax.fori_loop(1, L_INSTALL // NACC_I, inst_group, 0)
    for k in range(NACC_I):
        pltpu.make_async_copy(table_hbm.at[slab_i.at[k]], acc_i.at[k],
                              sems_i[k]).wait()

    for k in range(NACC_E):
        pltpu.async_copy(table_hbm.at[slab_e.at[k]], acc_e.at[k], sems_e[k])

    def ecc_group(g, carry):
        for k in range(NACC_E):
            j = g * NACC_E + k
            pltpu.make_async_copy(table_hbm.at[slab_e.at[j]], acc_e.at[k],
                                  sems_e[k]).wait()
            pltpu.async_copy(table_hbm.at[slab_e.at[j]], acc_e.at[k],
                             sems_e[k], add=True)
        return carry

    lax.fori_loop(1, L_ECC // NACC_E, ecc_group, 0)

    def comb_i(i, carry):
        for c in range(2):
            s = acc_i[0, i, pl.ds(16 * c, 16)]
            for k in range(1, NACC_I):
                s = s + acc_i[k, i, pl.ds(16 * c, 16)]
            out_v[i, pl.ds(16 * c, 16)] = s * (1.0 / L_INSTALL)
        return carry

    lax.fori_loop(0, BPW, comb_i, 0)

    for k in range(NACC_E):
        pltpu.make_async_copy(table_hbm.at[slab_e.at[k]], acc_e.at[k],
                              sems_e[k]).wait()

    def comb_e(i, carry):
        for c in range(2):
            s = acc_e[0, i, pl.ds(16 * c, 16)]
            for k in range(1, NACC_E):
                s = s + acc_e[k, i, pl.ds(16 * c, 16)]
            out_v[i, pl.ds(32 + 16 * c, 16)] = s * (1.0 / L_ECC)
        return carry

    lax.fori_loop(0, BPW, comb_e, 0)
    pltpu.sync_copy(out_v, out_hbm.at[pl.ds(base, BPW)])


def _permute_rows(r):
    return (r & ~jnp.int32(511)) + ((r & 127) << 2) + ((r >> 7) & 3)


@jax.jit
def kernel(install, install_ecc, app_table):
    inst_t = _permute_rows(install.astype(jnp.int32).T)
    ecc_t = _permute_rows(install_ecc.astype(jnp.int32).T)

    packed = pl.pallas_call(
        _tc_pack_body,
        grid=(PACK_GRID,),
        in_specs=[pl.BlockSpec((APP_DIM, PACK_W), lambda j: (0, j))],
        out_specs=pl.BlockSpec((4 * PACK_W // 16, 128), lambda j: (j, 0)),
        out_shape=jax.ShapeDtypeStruct((PACK_GRID * PACK_W // 4, 128),
                                       jnp.float32),
    )(app_table.T)
    tbl = packed.reshape(APP_SIZE_PAD, APP_DIM)

    mesh = plsc.VectorSubcoreMesh(core_axis_name="c", subcore_axis_name="s")
    run = pl.kernel(
        _sc_body,
        mesh=mesh,
        out_type=jax.ShapeDtypeStruct((B, 2 * APP_DIM), jnp.float32),
        scratch_types=[
            pltpu.VMEM((L_INSTALL, BPW), jnp.int32),
            pltpu.VMEM((L_ECC, BPW), jnp.int32),
            pltpu.VMEM((NACC_I, BPW, APP_DIM), jnp.float32),
            pltpu.VMEM((NACC_E, BPW, APP_DIM), jnp.float32),
            pltpu.VMEM((BPW, 2 * APP_DIM), jnp.float32),
        ] + [pltpu.SemaphoreType.DMA] * (NACC_I + NACC_E),
        compiler_params=pltpu.CompilerParams(use_tc_tiling_on_sc=False),
    )
    return run(inst_t, ecc_t, tbl)


# speedup vs baseline: 3.1679x; 1.2078x over previous
import jax
import jax.numpy as jnp
from jax import lax
from jax.experimental import pallas as pl
from jax.experimental.pallas import tpu as pltpu
from jax.experimental.pallas import tpu_sc as plsc

B = 4096
APP_DIM = 32
L_INSTALL = 200
L_ECC = 50
NC = 2
NS = 16
NW = NC * NS
BPW = B // NW
NACC_I = 8
NACC_E = 5

PACK_W = 16384
PACK_GRID = 62
APP_SIZE_PAD = PACK_GRID * PACK_W


def _tc_pack_body(x_ref, o_ref):
    x = x_ref[...]
    for s in range(PACK_W // 512):
        y = jnp.concatenate(
            [x[:, 128 * (4 * s + a):128 * (4 * s + a) + 128] for a in range(4)],
            axis=0)
        o_ref[pl.ds(128 * s, 128), :] = jnp.transpose(y)


def _sc_body(inst_t_hbm, ecc_t_hbm, table_hbm, out_hbm,
             slab_i, slab_e, acc_i, acc_e, out_v, *sems):
    sems_i = sems[:NACC_I]
    sems_e = sems[NACC_I:]
    wid = lax.axis_index("s") * NC + lax.axis_index("c")
    base = wid * BPW

    pltpu.sync_copy(inst_t_hbm.at[:, pl.ds(base, BPW)], slab_i)
    pltpu.sync_copy(ecc_t_hbm.at[:, pl.ds(base, BPW)], slab_e)

    for k in range(NACC_I):
        pltpu.async_copy(table_hbm.at[slab_i.at[k]], acc_i.at[k], sems_i[k])

    def inst_group(g, carry):
        for k in range(NACC_I):
            j = g * NACC_I + k
            pltpu.make_async_copy(table_hbm.at[slab_i.at[j]], acc_i.at[k],
                                  sems_i[k]).wait()
            pltpu.async_copy(table_hbm.at[slab_i.at[j]], acc_i.at[k],
                             sems_i[k], add=True)
        return carry

    lax.fori_loop(1, L_INSTALL // NACC_I, inst_group, 0)
    for k in range(NACC_I):
        pltpu.make_async_copy(table_hbm.at[slab_i.at[k]], acc_i.at[k],
                              sems_i[k]).wait()

    for k in range(NACC_E):
        pltpu.async_copy(table_hbm.at[slab_e.at[k]], acc_e.at[k], sems_e[k])

    def ecc_group(g, carry):
        for k in range(NACC_E):
            j = g * NACC_E + k
            pltpu.make_async_copy(table_hbm.at[slab_e.at[j]], acc_e.at[k],
                                  sems_e[k]).wait()
            pltpu.async_copy(table_hbm.at[slab_e.at[j]], acc_e.at[k],
                             sems_e[k], add=True)
        return carry

    lax.fori_loop(1, L_ECC // NACC_E, ecc_group, 0)

    def comb_i(i, carry):
        for c in range(2):
            s = acc_i[0, i, pl.ds(16 * c, 16)]
            for k in range(1, NACC_I):
                s = s + acc_i[k, i, pl.ds(16 * c, 16)]
            out_v[i, pl.ds(16 * c, 16)] = s * (1.0 / L_INSTALL)
        return carry

    lax.fori_loop(0, BPW, comb_i, 0)

    for k in range(NACC_E):
        pltpu.make_async_copy(table_hbm.at[slab_e.at[k]], acc_e.at[k],
                              sems_e[k]).wait()

    def comb_e(i, carry):
        for c in range(2):
            s = acc_e[0, i, pl.ds(16 * c, 16)]
            for k in range(1, NACC_E):
                s = s + acc_e[k, i, pl.ds(16 * c, 16)]
            out_v[i, pl.ds(32 + 16 * c, 16)] = s * (1.0 / L_ECC)
        return carry

    lax.fori_loop(0, BPW, comb_e, 0)
    pltpu.sync_copy(out_v, out_hbm.at[pl.ds(base, BPW)])


def _permute_rows(r):
    return (r & ~jnp.int32(511)) + ((r & 127) << 2) + ((r >> 7) & 3)


@jax.jit
def kernel(install, install_ecc, app_table):
    inst_t = _permute_rows(install.astype(jnp.int32).T)
    ecc_t = _permute_rows(install_ecc.astype(jnp.int32).T)

    packed = pl.pallas_call(
        _tc_pack_body,
        grid=(PACK_GRID,),
        in_specs=[pl.BlockSpec((APP_DIM, PACK_W), lambda j: (0, j))],
        out_specs=pl.BlockSpec((4 * PACK_W // 16, 128), lambda j: (j, 0)),
        out_shape=jax.ShapeDtypeStruct((PACK_GRID * PACK_W // 4, 128),
                                       jnp.float32),
    )(app_table.T)
    tbl = packed.reshape(APP_SIZE_PAD, APP_DIM)

    mesh = plsc.VectorSubcoreMesh(core_axis_name="c", subcore_axis_name="s")
    run = pl.kernel(
        _sc_body,
        mesh=mesh,
        out_type=jax.ShapeDtypeStruct((B, 2 * APP_DIM), jnp.float32),
        scratch_types=[
            pltpu.VMEM((L_INSTALL, BPW), jnp.int32),
            pltpu.VMEM((L_ECC, BPW), jnp.int32),
            pltpu.VMEM((NACC_I, BPW, APP_DIM), jnp.float32),
            pltpu.VMEM((NACC_E, BPW, APP_DIM), jnp.float32),
            pltpu.VMEM((BPW, 2 * APP_DIM), jnp.float32),
        ] + [pltpu.SemaphoreType.DMA] * (NACC_I + NACC_E),
        compiler_params=pltpu.CompilerParams(use_tc_tiling_on_sc=False),
    )
    return run(inst_t, ecc_t, tbl)


# speedup vs baseline: 3.4410x; 1.0862x over previous
import jax
import jax.numpy as jnp
from jax import lax
from jax.experimental import pallas as pl
from jax.experimental.pallas import tpu as pltpu
from jax.experimental.pallas import tpu_sc as plsc

B = 4096
APP_DIM = 32
L_INSTALL = 200
L_ECC = 50
NC = 2
NS = 16
NW = NC * NS
BPW = B // NW
NACC_I = 8
NACC_E = 5

PACK_W = 32768
PACK_GRID = 31
APP_SIZE_PAD = PACK_GRID * PACK_W


def _tc_pack_body(x_ref, o_ref):
    x = x_ref[...]
    for s in range(PACK_W // 512):
        y = jnp.concatenate(
            [x[:, 128 * (4 * s + a):128 * (4 * s + a) + 128] for a in range(4)],
            axis=0)
        o_ref[pl.ds(128 * s, 128), :] = jnp.transpose(y)


def _sc_body(inst_t_hbm, ecc_t_hbm, table_hbm, out_hbm,
             slab_i, slab_e, acc_i, acc_e, out_v, *sems):
    sems_i = sems[:NACC_I]
    sems_e = sems[NACC_I:]
    wid = lax.axis_index("s") * NC + lax.axis_index("c")
    base = wid * BPW

    pltpu.sync_copy(inst_t_hbm.at[:, pl.ds(base, BPW)], slab_i)
    pltpu.sync_copy(ecc_t_hbm.at[:, pl.ds(base, BPW)], slab_e)

    for k in range(NACC_I):
        pltpu.async_copy(table_hbm.at[slab_i.at[k]], acc_i.at[k], sems_i[k])

    def inst_group(g, carry):
        for k in range(NACC_I):
            j = g * NACC_I + k
            pltpu.make_async_copy(table_hbm.at[slab_i.at[j]], acc_i.at[k],
                                  sems_i[k]).wait()
            pltpu.async_copy(table_hbm.at[slab_i.at[j]], acc_i.at[k],
                             sems_i[k], add=True)
        return carry

    lax.fori_loop(1, L_INSTALL // NACC_I, inst_group, 0)
    for k in range(NACC_I):
        pltpu.make_async_copy(table_hbm.at[slab_i.at[k]], acc_i.at[k],
                              sems_i[k]).wait()

    for k in range(NACC_E):
        pltpu.async_copy(table_hbm.at[slab_e.at[k]], acc_e.at[k], sems_e[k])

    def ecc_group(g, carry):
        for k in range(NACC_E):
            j = g * NACC_E + k
            pltpu.make_async_copy(table_hbm.at[slab_e.at[j]], acc_e.at[k],
                                  sems_e[k]).wait()
            pltpu.async_copy(table_hbm.at[slab_e.at[j]], acc_e.at[k],
                             sems_e[k], add=True)
        return carry

    lax.fori_loop(1, L_ECC // NACC_E, ecc_group, 0)

    def comb_i(i, carry):
        for c in range(2):
            s = acc_i[0, i, pl.ds(16 * c, 16)]
            for k in range(1, NACC_I):
                s = s + acc_i[k, i, pl.ds(16 * c, 16)]
            out_v[i, pl.ds(16 * c, 16)] = s * (1.0 / L_INSTALL)
        return carry

    lax.fori_loop(0, BPW, comb_i, 0)

    for k in range(NACC_E):
        pltpu.make_async_copy(table_hbm.at[slab_e.at[k]], acc_e.at[k],
                              sems_e[k]).wait()

    def comb_e(i, carry):
        for c in range(2):
            s = acc_e[0, i, pl.ds(16 * c, 16)]
            for k in range(1, NACC_E):
                s = s + acc_e[k, i, pl.ds(16 * c, 16)]
            out_v[i, pl.ds(32 + 16 * c, 16)] = s * (1.0 / L_ECC)
        return carry

    lax.fori_loop(0, BPW, comb_e, 0)
    pltpu.sync_copy(out_v, out_hbm.at[pl.ds(base, BPW)])


def _permute_rows(r):
    return (r & ~jnp.int32(511)) + ((r & 127) << 2) + ((r >> 7) & 3)


@jax.jit
def kernel(install, install_ecc, app_table):
    inst_t = _permute_rows(install.astype(jnp.int32).T)
    ecc_t = _permute_rows(install_ecc.astype(jnp.int32).T)

    packed = pl.pallas_call(
        _tc_pack_body,
        grid=(PACK_GRID,),
        in_specs=[pl.BlockSpec((APP_DIM, PACK_W), lambda j: (0, j))],
        out_specs=pl.BlockSpec((4 * PACK_W // 16, 128), lambda j: (j, 0)),
        out_shape=jax.ShapeDtypeStruct((PACK_GRID * PACK_W // 4, 128),
                                       jnp.float32),
    )(app_table.T)
    tbl = packed.reshape(APP_SIZE_PAD, APP_DIM)

    mesh = plsc.VectorSubcoreMesh(core_axis_name="c", subcore_axis_name="s")
    run = pl.kernel(
        _sc_body,
        mesh=mesh,
        out_type=jax.ShapeDtypeStruct((B, 2 * APP_DIM), jnp.float32),
        scratch_types=[
            pltpu.VMEM((L_INSTALL, BPW), jnp.int32),
            pltpu.VMEM((L_ECC, BPW), jnp.int32),
            pltpu.VMEM((NACC_I, BPW, APP_DIM), jnp.float32),
            pltpu.VMEM((NACC_E, BPW, APP_DIM), jnp.float32),
            pltpu.VMEM((BPW, 2 * APP_DIM), jnp.float32),
        ] + [pltpu.SemaphoreType.DMA] * (NACC_I + NACC_E),
        compiler_params=pltpu.CompilerParams(use_tc_tiling_on_sc=False),
    )
    return run(inst_t, ecc_t, tbl)


# speedup vs baseline: 3.4649x; 1.0070x over previous
import jax
import jax.numpy as jnp
from jax import lax
from jax.experimental import pallas as pl
from jax.experimental.pallas import tpu as pltpu
from jax.experimental.pallas import tpu_sc as plsc

B = 4096
APP_DIM = 32
L_INSTALL = 200
L_ECC = 50
NC = 2
NS = 16
NW = NC * NS
BPW = B // NW
NACC_I = 8
NACC_E = 5

PACK_W = 65536
PACK_GRID = 16
APP_SIZE_PAD = PACK_GRID * PACK_W


def _tc_pack_body(x_ref, o_ref):
    x = x_ref[...]
    for s in range(PACK_W // 512):
        y = jnp.concatenate(
            [x[:, 128 * (4 * s + a):128 * (4 * s + a) + 128] for a in range(4)],
            axis=0)
        o_ref[pl.ds(128 * s, 128), :] = jnp.transpose(y)


def _sc_body(inst_t_hbm, ecc_t_hbm, table_hbm, out_hbm,
             slab_i, slab_e, acc_i, acc_e, out_v, *sems):
    sems_i = sems[:NACC_I]
    sems_e = sems[NACC_I:]
    wid = lax.axis_index("s") * NC + lax.axis_index("c")
    base = wid * BPW

    pltpu.sync_copy(inst_t_hbm.at[:, pl.ds(base, BPW)], slab_i)
    pltpu.sync_copy(ecc_t_hbm.at[:, pl.ds(base, BPW)], slab_e)

    for k in range(NACC_I):
        pltpu.async_copy(table_hbm.at[slab_i.at[k]], acc_i.at[k], sems_i[k])

    def inst_group(g, carry):
        for k in range(NACC_I):
            j = g * NACC_I + k
            pltpu.make_async_copy(table_hbm.at[slab_i.at[j]], acc_i.at[k],
                                  sems_i[k]).wait()
            pltpu.async_copy(table_hbm.at[slab_i.at[j]], acc_i.at[k],
                             sems_i[k], add=True)
        return carry

    lax.fori_loop(1, L_INSTALL // NACC_I, inst_group, 0)
    for k in range(NACC_I):
        pltpu.make_async_copy(table_hbm.at[slab_i.at[k]], acc_i.at[k],
                              sems_i[k]).wait()

    for k in range(NACC_E):
        pltpu.async_copy(table_hbm.at[slab_e.at[k]], acc_e.at[k], sems_e[k])

    def ecc_group(g, carry):
        for k in range(NACC_E):
            j = g * NACC_E + k
            pltpu.make_async_copy(table_hbm.at[slab_e.at[j]], acc_e.at[k],
                                  sems_e[k]).wait()
            pltpu.async_copy(table_hbm.at[slab_e.at[j]], acc_e.at[k],
                             sems_e[k], add=True)
        return carry

    lax.fori_loop(1, L_ECC // NACC_E, ecc_group, 0)

    def comb_i(i, carry):
        for c in range(2):
            s = acc_i[0, i, pl.ds(16 * c, 16)]
            for k in range(1, NACC_I):
                s = s + acc_i[k, i, pl.ds(16 * c, 16)]
            out_v[i, pl.ds(16 * c, 16)] = s * (1.0 / L_INSTALL)
        return carry

    lax.fori_loop(0, BPW, comb_i, 0)

    for k in range(NACC_E):
        pltpu.make_async_copy(table_hbm.at[slab_e.at[k]], acc_e.at[k],
                              sems_e[k]).wait()

    def comb_e(i, carry):
        for c in range(2):
            s = acc_e[0, i, pl.ds(16 * c, 16)]
            for k in range(1, NACC_E):
                s = s + acc_e[k, i, pl.ds(16 * c, 16)]
            out_v[i, pl.ds(32 + 16 * c, 16)] = s * (1.0 / L_ECC)
        return carry

    lax.fori_loop(0, BPW, comb_e, 0)
    pltpu.sync_copy(out_v, out_hbm.at[pl.ds(base, BPW)])


def _permute_rows(r):
    return (r & ~jnp.int32(511)) + ((r & 127) << 2) + ((r >> 7) & 3)


@jax.jit
def kernel(install, install_ecc, app_table):
    inst_t = _permute_rows(install.astype(jnp.int32).T)
    ecc_t = _permute_rows(install_ecc.astype(jnp.int32).T)

    packed = pl.pallas_call(
        _tc_pack_body,
        grid=(PACK_GRID,),
        in_specs=[pl.BlockSpec((APP_DIM, PACK_W), lambda j: (0, j))],
        out_specs=pl.BlockSpec((4 * PACK_W // 16, 128), lambda j: (j, 0)),
        out_shape=jax.ShapeDtypeStruct((PACK_GRID * PACK_W // 4, 128),
                                       jnp.float32),
    )(app_table.T)
    tbl = packed.reshape(APP_SIZE_PAD, APP_DIM)

    mesh = plsc.VectorSubcoreMesh(core_axis_name="c", subcore_axis_name="s")
    run = pl.kernel(
        _sc_body,
        mesh=mesh,
        out_type=jax.ShapeDtypeStruct((B, 2 * APP_DIM), jnp.float32),
        scratch_types=[
            pltpu.VMEM((L_INSTALL, BPW), jnp.int32),
            pltpu.VMEM((L_ECC, BPW), jnp.int32),
            pltpu.VMEM((NACC_I, BPW, APP_DIM), jnp.float32),
            pltpu.VMEM((NACC_E, BPW, APP_DIM), jnp.float32),
            pltpu.VMEM((BPW, 2 * APP_DIM), jnp.float32),
        ] + [pltpu.SemaphoreType.DMA] * (NACC_I + NACC_E),
        compiler_params=pltpu.CompilerParams(use_tc_tiling_on_sc=False),
    )
    return run(inst_t, ecc_t, tbl)
